# Initial kernel scaffold; baseline (speedup 1.0000x reference)
import jax
import jax.numpy as jnp
from jax.experimental import pallas as pl


def kernel(x, edge_index, W1, b1, g1, bb1, W2, b2, g2, bb2, W3, b3, W4, b4):
    raise NotImplementedError("write your pallas kernel here")


# speedup vs baseline: 13.8958x; 13.8958x over previous
import functools

import jax
import jax.numpy as jnp
from jax import lax
from jax.experimental import pallas as pl
from jax.experimental.pallas import tpu as pltpu
from jax.experimental.pallas import tpu_sc as plsc

N_NODES = 100000
N_PAD = 102400
E_EDGES = 6400000
EL = 128
R_ROWS = E_EDGES // EL
N_SC = 2
N_TILES = 16
N_WORKERS = N_SC * N_TILES
TILE_ROWS = N_PAD // N_TILES
RB = 800
FW = 16

@functools.lru_cache(maxsize=None)
def _sc_mesh():
    return plsc.VectorSubcoreMesh(
        core_axis_name="c", subcore_axis_name="s", num_cores=N_SC,
        num_subcores=N_TILES)


def _worker_range(wid, total, workers):
    q, r = total // workers, total % workers
    nrows = jnp.where(wid < r, q + 1, q)
    base = wid * q + jnp.minimum(wid, r)
    return base, nrows


def _deg_body(dst_h, ones_h, zeros_h, out_h, onesv, dstv, acc, obuf):
    c = lax.axis_index("c")
    s = lax.axis_index("s")
    wid = c * N_TILES + s
    pltpu.sync_copy(ones_h, onesv)
    pltpu.sync_copy(zeros_h, obuf)
    pltpu.sync_copy(obuf, acc.at[pl.ds(s * TILE_ROWS, TILE_ROWS)])
    plsc.subcore_barrier()
    base, nrows = _worker_range(wid, R_ROWS, N_WORKERS)

    def body(j, carry):
        pltpu.sync_copy(dst_h.at[base + j], dstv)
        pltpu.sync_copy(onesv, acc.at[dstv], add=True)
        return carry

    lax.fori_loop(0, nrows, body, 0)
    plsc.subcore_barrier()
    sl = pl.ds(s * TILE_ROWS, TILE_ROWS)
    pltpu.sync_copy(acc.at[sl], obuf)
    pltpu.sync_copy(obuf, out_h.at[c, sl])


@functools.lru_cache(maxsize=None)
def _deg_call():
    return pl.kernel(
        _deg_body,
        out_type=jax.ShapeDtypeStruct((N_SC, N_PAD), jnp.float32),
        mesh=_sc_mesh(),
        compiler_params=pltpu.CompilerParams(use_tc_tiling_on_sc=False),
        scratch_types=[
            pltpu.VMEM((EL,), jnp.float32),
            pltpu.VMEM((EL,), jnp.int32),
            pltpu.VMEM_SHARED((N_PAD,), jnp.float32),
            pltpu.VMEM((TILE_ROWS,), jnp.float32),
        ],
    )


def _pass_body(col_split, src_h, dst_h, t0_h, t1_h, zeros_h, out_h,
               srcv, dstv, rowsv, acc, obuf, sem):
    c = lax.axis_index("c")
    s = lax.axis_index("s")
    pltpu.sync_copy(zeros_h, obuf)
    for i in range(TILE_ROWS // RB):
        pltpu.sync_copy(obuf, acc.at[pl.ds(s * TILE_ROWS + i * RB, RB)])
    plsc.subcore_barrier()

    if col_split:
        base = s * (R_ROWS // N_TILES)
        nrows = R_ROWS // N_TILES
    else:
        base, nrows = _worker_range(c * N_TILES + s, R_ROWS, N_WORKERS)

    def run(tbl):
        def body(j, carry):
            row = base + j
            pltpu.sync_copy(src_h.at[row], srcv)
            pltpu.sync_copy(dst_h.at[row], dstv)
            pltpu.async_copy(tbl.at[srcv], rowsv, sem).wait()
            pltpu.sync_copy(rowsv, acc.at[dstv], add=True)
            return carry
        lax.fori_loop(0, nrows, body, 0)

    @pl.when(c == 0)
    def _():
        run(t0_h)

    @pl.when(c == 1)
    def _():
        run(t1_h)

    plsc.subcore_barrier()
    for i in range(TILE_ROWS // RB):
        sl = pl.ds(s * TILE_ROWS + i * RB, RB)
        pltpu.sync_copy(acc.at[sl], obuf)
        pltpu.sync_copy(obuf, out_h.at[c, sl])


@functools.lru_cache(maxsize=None)
def _make_pass(col_split):
    return pl.kernel(
        functools.partial(_pass_body, col_split),
        out_type=jax.ShapeDtypeStruct((N_SC, N_PAD, FW), jnp.float32),
        mesh=_sc_mesh(),
        compiler_params=pltpu.CompilerParams(use_tc_tiling_on_sc=False),
        scratch_types=[
            pltpu.VMEM((EL,), jnp.int32),
            pltpu.VMEM((EL,), jnp.int32),
            pltpu.VMEM((EL, FW), jnp.float32),
            pltpu.VMEM_SHARED((N_PAD, FW), jnp.float32),
            pltpu.VMEM((RB, FW), jnp.float32),
            pltpu.SemaphoreType.DMA,
        ],
    )


BN = 4000
_GRID = N_NODES // BN


def _row_spec(w):
    return pl.BlockSpec((BN, w), lambda i: (i, 0))


def _full_spec(shape):
    return pl.BlockSpec(shape, lambda i: (0, 0))


def _tc0_body(d0, d1, xp, dis, t1):
    deg = d0[...] + d1[...] + 1.0
    dv = lax.rsqrt(deg)
    dis[...] = dv
    t1[...] = dv * xp[...]


_tc0 = pl.pallas_call(
    _tc0_body,
    grid=(_GRID,),
    in_specs=[_row_spec(1), _row_spec(1), _row_spec(FW)],
    out_specs=[_row_spec(1), _row_spec(FW)],
    out_shape=[jax.ShapeDtypeStruct((N_NODES, 1), jnp.float32),
               jax.ShapeDtypeStruct((N_NODES, FW), jnp.float32)],
)


def _tc1_body(dis, p0, p1, t1, w1, b1, w2, t2a, t2b):
    d = dis[...]
    z = d * (p0[...] + p1[...] + t1[...])
    h1 = jnp.maximum(
        jnp.dot(z, w1[...], preferred_element_type=jnp.float32) + b1[...], 0.0)
    t2 = d * jnp.dot(h1, w2[...], preferred_element_type=jnp.float32)
    t2a[...] = t2[:, :FW]
    t2b[...] = t2[:, FW:]


_tc1 = pl.pallas_call(
    _tc1_body,
    grid=(_GRID,),
    in_specs=[_row_spec(1), _row_spec(FW), _row_spec(FW), _row_spec(FW),
              _full_spec((FW, 64)), _full_spec((1, 64)), _full_spec((64, 32))],
    out_specs=[_row_spec(FW), _row_spec(FW)],
    out_shape=[jax.ShapeDtypeStruct((N_NODES, FW), jnp.float32),
               jax.ShapeDtypeStruct((N_NODES, FW), jnp.float32)],
)


def _tc2_body(dis, q0, q1, t2a, t2b, g2, b2, w3, t3):
    d = dis[...]
    za = d * (q0[...] + t2a[...])
    zb = d * (q1[...] + t2b[...])
    z = jnp.concatenate([za, zb], axis=1)
    h2 = jnp.maximum(z * g2[...] + b2[...], 0.0)
    t3[...] = d * jnp.dot(h2, w3[...], preferred_element_type=jnp.float32)


_tc2 = pl.pallas_call(
    _tc2_body,
    grid=(_GRID,),
    in_specs=[_row_spec(1), _row_spec(FW), _row_spec(FW), _row_spec(FW),
              _row_spec(FW), _full_spec((1, 32)), _full_spec((1, 32)),
              _full_spec((32, FW))],
    out_specs=_row_spec(FW),
    out_shape=jax.ShapeDtypeStruct((N_NODES, FW), jnp.float32),
)


def _tc3_body(dis, p0, p1, t3, b3, w4, t4):
    d = dis[...]
    h3 = jnp.maximum(d * (p0[...] + p1[...] + t3[...]) + b3[...], 0.0)
    t4[...] = d * jnp.dot(h3, w4[...], preferred_element_type=jnp.float32)


_tc3 = pl.pallas_call(
    _tc3_body,
    grid=(_GRID,),
    in_specs=[_row_spec(1), _row_spec(FW), _row_spec(FW), _row_spec(FW),
              _full_spec((1, FW)), _full_spec((FW, FW))],
    out_specs=_row_spec(FW),
    out_shape=jax.ShapeDtypeStruct((N_NODES, FW), jnp.float32),
)


def _tc4_body(dis, p0, p1, t4, b4, out):
    out[...] = dis[...] * (p0[...] + p1[...] + t4[...]) + b4[...]


_tc4 = pl.pallas_call(
    _tc4_body,
    grid=(_GRID,),
    in_specs=[_row_spec(1), _row_spec(FW), _row_spec(FW), _row_spec(FW),
              _full_spec((1, FW))],
    out_specs=_row_spec(FW),
    out_shape=jax.ShapeDtypeStruct((N_NODES, FW), jnp.float32),
)


def kernel(x, edge_index, W1, b1, g1, bb1, W2, b2, g2, bb2, W3, b3, W4, b4):
    src2 = edge_index[0].reshape(R_ROWS, EL)
    dst2 = edge_index[1].reshape(R_ROWS, EL)
    ones = jnp.ones((EL,), jnp.float32)
    zeros1 = jnp.zeros((TILE_ROWS,), jnp.float32)
    zeros2 = jnp.zeros((RB, FW), jnp.float32)

    degp = _deg_call()(dst2, ones, zeros1)
    d0 = degp[0, :N_NODES, None]
    d1 = degp[1, :N_NODES, None]
    xp = jnp.pad(x, ((0, 0), (0, FW - x.shape[1])))
    dis, t1 = _tc0(d0, d1, xp)

    p = _make_pass(False)(src2, dst2, t1, t1, zeros2)
    cbn = (1.0 / jnp.sqrt(1.0 + 1e-5)).astype(jnp.float32)
    W1e = jnp.pad(W1, ((0, FW - W1.shape[0]), (0, 0))) * (cbn * g1)[None, :]
    b1e = (b1 * cbn * g1 + bb1)[None, :]
    t2a, t2b = _tc1(dis, p[0, :N_NODES], p[1, :N_NODES], t1, W1e, b1e, W2)

    q = _make_pass(True)(src2, dst2, t2a, t2b, zeros2)
    g2v = (cbn * g2)[None, :]
    b2e = (b2 * cbn * g2 + bb2)[None, :]
    t3 = _tc2(dis, q[0, :N_NODES], q[1, :N_NODES], t2a, t2b, g2v, b2e, W3)

    p = _make_pass(False)(src2, dst2, t3, t3, zeros2)
    W4p = jnp.pad(W4, ((0, 0), (0, FW - W4.shape[1])))
    t4 = _tc3(dis, p[0, :N_NODES], p[1, :N_NODES], t3, b3[None, :], W4p)

    p = _make_pass(False)(src2, dst2, t4, t4, zeros2)
    b4p = jnp.pad(b4, (0, FW - b4.shape[0]))[None, :]
    out16 = _tc4(dis, p[0, :N_NODES], p[1, :N_NODES], t4, b4p)
    return out16[:, :W4.shape[1]]


# speedup vs baseline: 50.7165x; 3.6498x over previous
import functools

import jax
import jax.numpy as jnp
from jax import lax
from jax.experimental import pallas as pl
from jax.experimental.pallas import tpu as pltpu
from jax.experimental.pallas import tpu_sc as plsc

N_NODES = 100000
N_PAD = 102400
E_EDGES = 6400000
EL = 128
R_ROWS = E_EDGES // EL
N_SC = 2
N_TILES = 16
N_WORKERS = N_SC * N_TILES
TILE_ROWS = N_PAD // N_TILES
RB = 800
FW = 16

@functools.lru_cache(maxsize=None)
def _sc_mesh():
    return plsc.VectorSubcoreMesh(
        core_axis_name="c", subcore_axis_name="s", num_cores=N_SC,
        num_subcores=N_TILES)


def _worker_range(wid, total, workers):
    q, r = total // workers, total % workers
    nrows = jnp.where(wid < r, q + 1, q)
    base = wid * q + jnp.minimum(wid, r)
    return base, nrows


K = 12


def _deg_body(dst_h, ones_h, zeros_h, out_h, onesv, dstv, acc, obuf, ssem):
    c = lax.axis_index("c")
    s = lax.axis_index("s")
    wid = c * N_TILES + s
    pltpu.sync_copy(ones_h, onesv)
    pltpu.sync_copy(zeros_h, obuf)
    pltpu.sync_copy(obuf, acc.at[pl.ds(s * TILE_ROWS, TILE_ROWS)])
    plsc.subcore_barrier()
    base, nrows = _worker_range(wid, R_ROWS, N_WORKERS)
    nfull = nrows // K

    def chunk_body(ci, carry):
        cb = base + ci * K
        pltpu.sync_copy(dst_h.at[pl.ds(cb, K)], dstv)
        descs = [pltpu.async_copy(onesv, acc.at[dstv.at[j]], ssem, add=True)
                 for j in range(K)]
        for dd in descs:
            dd.wait()
        return carry

    lax.fori_loop(0, nfull, chunk_body, 0)

    def rem_body(j, carry):
        pltpu.sync_copy(dst_h.at[base + nfull * K + j], dstv.at[0])
        pltpu.sync_copy(onesv, acc.at[dstv.at[0]], add=True)
        return carry

    lax.fori_loop(0, nrows - nfull * K, rem_body, 0)
    plsc.subcore_barrier()
    sl = pl.ds(s * TILE_ROWS, TILE_ROWS)
    pltpu.sync_copy(acc.at[sl], obuf)
    pltpu.sync_copy(obuf, out_h.at[c, sl])


@functools.lru_cache(maxsize=None)
def _deg_call():
    return pl.kernel(
        _deg_body,
        out_type=jax.ShapeDtypeStruct((N_SC, N_PAD), jnp.float32),
        mesh=_sc_mesh(),
        compiler_params=pltpu.CompilerParams(use_tc_tiling_on_sc=False),
        scratch_types=[
            pltpu.VMEM((EL,), jnp.float32),
            pltpu.VMEM((K, EL), jnp.int32),
            pltpu.VMEM_SHARED((N_PAD,), jnp.float32),
            pltpu.VMEM((TILE_ROWS,), jnp.float32),
            pltpu.SemaphoreType.DMA,
        ],
    )


def _pass_body(col_split, src_h, dst_h, t0_h, t1_h, zeros_h, out_h,
               srcv, dstv, rowsv, acc, gsem, ssem):
    c = lax.axis_index("c")
    s = lax.axis_index("s")
    pltpu.sync_copy(zeros_h, acc.at[pl.ds(s * TILE_ROWS, TILE_ROWS)])
    plsc.subcore_barrier()

    if col_split:
        base = s * (R_ROWS // N_TILES)
        nrows = R_ROWS // N_TILES
    else:
        base, nrows = _worker_range(c * N_TILES + s, R_ROWS, N_WORKERS)
    nfull = nrows // K

    def run(tbl):
        def chunk_body(ci, carry):
            cb = base + ci * K
            pltpu.sync_copy(src_h.at[pl.ds(cb, K)], srcv)
            pltpu.sync_copy(dst_h.at[pl.ds(cb, K)], dstv)
            gds = [pltpu.async_copy(tbl.at[srcv.at[j]], rowsv.at[j], gsem)
                   for j in range(K)]
            for dd in gds:
                dd.wait()
            sds = [pltpu.async_copy(rowsv.at[j], acc.at[dstv.at[j]], ssem,
                                    add=True)
                   for j in range(K)]
            for dd in sds:
                dd.wait()
            return carry

        lax.fori_loop(0, nfull, chunk_body, 0)

        def rem_body(j, carry):
            row = base + nfull * K + j
            pltpu.sync_copy(src_h.at[row], srcv.at[0])
            pltpu.sync_copy(dst_h.at[row], dstv.at[0])
            pltpu.async_copy(tbl.at[srcv.at[0]], rowsv.at[0], gsem).wait()
            pltpu.sync_copy(rowsv.at[0], acc.at[dstv.at[0]], add=True)
            return carry

        lax.fori_loop(0, nrows - nfull * K, rem_body, 0)

    @pl.when(c == 0)
    def _():
        run(t0_h)

    @pl.when(c == 1)
    def _():
        run(t1_h)

    plsc.subcore_barrier()
    sl = pl.ds(s * TILE_ROWS, TILE_ROWS)
    pltpu.sync_copy(acc.at[sl], out_h.at[c, sl])


@functools.lru_cache(maxsize=None)
def _make_pass(col_split):
    return pl.kernel(
        functools.partial(_pass_body, col_split),
        out_type=jax.ShapeDtypeStruct((N_SC, N_PAD, FW), jnp.float32),
        mesh=_sc_mesh(),
        compiler_params=pltpu.CompilerParams(use_tc_tiling_on_sc=False),
        scratch_types=[
            pltpu.VMEM((K, EL), jnp.int32),
            pltpu.VMEM((K, EL), jnp.int32),
            pltpu.VMEM((K, EL, FW), jnp.float32),
            pltpu.VMEM_SHARED((N_PAD, FW), jnp.float32),
            pltpu.SemaphoreType.DMA,
            pltpu.SemaphoreType.DMA,
        ],
    )


BN = 4000
_GRID = N_NODES // BN


def _row_spec(w):
    return pl.BlockSpec((BN, w), lambda i: (i, 0))


def _full_spec(shape):
    return pl.BlockSpec(shape, lambda i: (0, 0))


def _tc0_body(d0, d1, xp, dis, t1):
    deg = d0[...] + d1[...] + 1.0
    dv = lax.rsqrt(deg)
    dis[...] = dv
    t1[...] = dv * xp[...]


_tc0 = pl.pallas_call(
    _tc0_body,
    grid=(_GRID,),
    in_specs=[_row_spec(1), _row_spec(1), _row_spec(FW)],
    out_specs=[_row_spec(1), _row_spec(FW)],
    out_shape=[jax.ShapeDtypeStruct((N_NODES, 1), jnp.float32),
               jax.ShapeDtypeStruct((N_NODES, FW), jnp.float32)],
)


def _tc1_body(dis, p0, p1, t1, w1, b1, w2, t2a, t2b):
    d = dis[...]
    z = d * (p0[...] + p1[...] + t1[...])
    h1 = jnp.maximum(
        jnp.dot(z, w1[...], preferred_element_type=jnp.float32) + b1[...], 0.0)
    t2 = d * jnp.dot(h1, w2[...], preferred_element_type=jnp.float32)
    t2a[...] = t2[:, :FW]
    t2b[...] = t2[:, FW:]


_tc1 = pl.pallas_call(
    _tc1_body,
    grid=(_GRID,),
    in_specs=[_row_spec(1), _row_spec(FW), _row_spec(FW), _row_spec(FW),
              _full_spec((FW, 64)), _full_spec((1, 64)), _full_spec((64, 32))],
    out_specs=[_row_spec(FW), _row_spec(FW)],
    out_shape=[jax.ShapeDtypeStruct((N_NODES, FW), jnp.float32),
               jax.ShapeDtypeStruct((N_NODES, FW), jnp.float32)],
)


def _tc2_body(dis, q0, q1, t2a, t2b, g2, b2, w3, t3):
    d = dis[...]
    za = d * (q0[...] + t2a[...])
    zb = d * (q1[...] + t2b[...])
    z = jnp.concatenate([za, zb], axis=1)
    h2 = jnp.maximum(z * g2[...] + b2[...], 0.0)
    t3[...] = d * jnp.dot(h2, w3[...], preferred_element_type=jnp.float32)


_tc2 = pl.pallas_call(
    _tc2_body,
    grid=(_GRID,),
    in_specs=[_row_spec(1), _row_spec(FW), _row_spec(FW), _row_spec(FW),
              _row_spec(FW), _full_spec((1, 32)), _full_spec((1, 32)),
              _full_spec((32, FW))],
    out_specs=_row_spec(FW),
    out_shape=jax.ShapeDtypeStruct((N_NODES, FW), jnp.float32),
)


def _tc3_body(dis, p0, p1, t3, b3, w4, t4):
    d = dis[...]
    h3 = jnp.maximum(d * (p0[...] + p1[...] + t3[...]) + b3[...], 0.0)
    t4[...] = d * jnp.dot(h3, w4[...], preferred_element_type=jnp.float32)


_tc3 = pl.pallas_call(
    _tc3_body,
    grid=(_GRID,),
    in_specs=[_row_spec(1), _row_spec(FW), _row_spec(FW), _row_spec(FW),
              _full_spec((1, FW)), _full_spec((FW, FW))],
    out_specs=_row_spec(FW),
    out_shape=jax.ShapeDtypeStruct((N_NODES, FW), jnp.float32),
)


def _tc4_body(dis, p0, p1, t4, b4, out):
    out[...] = dis[...] * (p0[...] + p1[...] + t4[...]) + b4[...]


_tc4 = pl.pallas_call(
    _tc4_body,
    grid=(_GRID,),
    in_specs=[_row_spec(1), _row_spec(FW), _row_spec(FW), _row_spec(FW),
              _full_spec((1, FW))],
    out_specs=_row_spec(FW),
    out_shape=jax.ShapeDtypeStruct((N_NODES, FW), jnp.float32),
)


def kernel(x, edge_index, W1, b1, g1, bb1, W2, b2, g2, bb2, W3, b3, W4, b4):
    src2 = edge_index[0].reshape(R_ROWS, EL)
    dst2 = edge_index[1].reshape(R_ROWS, EL)
    ones = jnp.ones((EL,), jnp.float32)
    zeros1 = jnp.zeros((TILE_ROWS,), jnp.float32)
    zeros2 = jnp.zeros((TILE_ROWS, FW), jnp.float32)

    degp = _deg_call()(dst2, ones, zeros1)
    d0 = degp[0, :N_NODES, None]
    d1 = degp[1, :N_NODES, None]
    xp = jnp.pad(x, ((0, 0), (0, FW - x.shape[1])))
    dis, t1 = _tc0(d0, d1, xp)

    p = _make_pass(False)(src2, dst2, t1, t1, zeros2)
    cbn = (1.0 / jnp.sqrt(1.0 + 1e-5)).astype(jnp.float32)
    W1e = jnp.pad(W1, ((0, FW - W1.shape[0]), (0, 0))) * (cbn * g1)[None, :]
    b1e = (b1 * cbn * g1 + bb1)[None, :]
    t2a, t2b = _tc1(dis, p[0, :N_NODES], p[1, :N_NODES], t1, W1e, b1e, W2)

    q = _make_pass(True)(src2, dst2, t2a, t2b, zeros2)
    g2v = (cbn * g2)[None, :]
    b2e = (b2 * cbn * g2 + bb2)[None, :]
    t3 = _tc2(dis, q[0, :N_NODES], q[1, :N_NODES], t2a, t2b, g2v, b2e, W3)

    p = _make_pass(False)(src2, dst2, t3, t3, zeros2)
    W4p = jnp.pad(W4, ((0, 0), (0, FW - W4.shape[1])))
    t4 = _tc3(dis, p[0, :N_NODES], p[1, :N_NODES], t3, b3[None, :], W4p)

    p = _make_pass(False)(src2, dst2, t4, t4, zeros2)
    b4p = jnp.pad(b4, (0, FW - b4.shape[0]))[None, :]
    out16 = _tc4(dis, p[0, :N_NODES], p[1, :N_NODES], t4, b4p)
    return out16[:, :W4.shape[1]]


# speedup vs baseline: 60.7743x; 1.1983x over previous
import functools

import jax
import jax.numpy as jnp
from jax import lax
from jax.experimental import pallas as pl
from jax.experimental.pallas import tpu as pltpu
from jax.experimental.pallas import tpu_sc as plsc

N_NODES = 100000
N_PAD = 102400
E_EDGES = 6400000
EL = 128
R_ROWS = E_EDGES // EL
N_SC = 2
N_TILES = 16
N_WORKERS = N_SC * N_TILES
TILE_ROWS = N_PAD // N_TILES
RB = 800
FW = 16

@functools.lru_cache(maxsize=None)
def _sc_mesh():
    return plsc.VectorSubcoreMesh(
        core_axis_name="c", subcore_axis_name="s", num_cores=N_SC,
        num_subcores=N_TILES)


def _worker_range(wid, total, workers):
    q, r = total // workers, total % workers
    nrows = jnp.where(wid < r, q + 1, q)
    base = wid * q + jnp.minimum(wid, r)
    return base, nrows


K = 12


def _deg_body(dst_h, ones_h, zeros_h, out_h, onesv, dstv, acc, obuf, ssem):
    c = lax.axis_index("c")
    s = lax.axis_index("s")
    wid = c * N_TILES + s
    pltpu.sync_copy(ones_h, onesv)
    pltpu.sync_copy(zeros_h, obuf)
    pltpu.sync_copy(obuf, acc.at[pl.ds(s * TILE_ROWS, TILE_ROWS)])
    plsc.subcore_barrier()
    base, nrows = _worker_range(wid, R_ROWS, N_WORKERS)
    nfull = nrows // K

    def chunk_body(ci, carry):
        cb = base + ci * K
        pltpu.sync_copy(dst_h.at[pl.ds(cb, K)], dstv)
        descs = [pltpu.async_copy(onesv, acc.at[dstv.at[j]], ssem, add=True)
                 for j in range(K)]
        for dd in descs:
            dd.wait()
        return carry

    lax.fori_loop(0, nfull, chunk_body, 0)

    def rem_body(j, carry):
        pltpu.sync_copy(dst_h.at[base + nfull * K + j], dstv.at[0])
        pltpu.sync_copy(onesv, acc.at[dstv.at[0]], add=True)
        return carry

    lax.fori_loop(0, nrows - nfull * K, rem_body, 0)
    plsc.subcore_barrier()
    sl = pl.ds(s * TILE_ROWS, TILE_ROWS)
    pltpu.sync_copy(acc.at[sl], obuf)
    pltpu.sync_copy(obuf, out_h.at[c, sl])


@functools.lru_cache(maxsize=None)
def _deg_call():
    return pl.kernel(
        _deg_body,
        out_type=jax.ShapeDtypeStruct((N_SC, N_PAD), jnp.float32),
        mesh=_sc_mesh(),
        compiler_params=pltpu.CompilerParams(use_tc_tiling_on_sc=False),
        scratch_types=[
            pltpu.VMEM((EL,), jnp.float32),
            pltpu.VMEM((K, EL), jnp.int32),
            pltpu.VMEM_SHARED((N_PAD,), jnp.float32),
            pltpu.VMEM((TILE_ROWS,), jnp.float32),
            pltpu.SemaphoreType.DMA,
        ],
    )


K2 = 6


def _pass_body(col_split, src_h, dst_h, t0_h, t1_h, zeros_h, out_h,
               srcv, dstv, rowsv, acc, gsemA, gsemB, ssemA, ssemB):
    c = lax.axis_index("c")
    s = lax.axis_index("s")
    pltpu.sync_copy(zeros_h, acc.at[pl.ds(s * TILE_ROWS, TILE_ROWS)])
    plsc.subcore_barrier()

    if col_split:
        base = s * (R_ROWS // N_TILES)
        nrows = R_ROWS // N_TILES
    else:
        base, nrows = _worker_range(c * N_TILES + s, R_ROWS, N_WORKERS)
    nfull = nrows // K2
    npair = nfull // 2

    def run(tbl):
        sems = [(gsemA, ssemA), (gsemB, ssemB)]

        def fire(b, cb):
            pltpu.sync_copy(src_h.at[pl.ds(cb, K2)], srcv.at[b])
            pltpu.sync_copy(dst_h.at[pl.ds(cb, K2)], dstv.at[b])
            for j in range(K2):
                pltpu.async_copy(tbl.at[srcv.at[b].at[j]],
                                 rowsv.at[b].at[j], sems[b][0])

        def drain_and_scatter(b):
            for j in range(K2):
                pltpu.make_async_copy(tbl.at[srcv.at[b].at[j]],
                                      rowsv.at[b].at[j], sems[b][0]).wait()
            sds = [pltpu.async_copy(rowsv.at[b].at[j],
                                    acc.at[dstv.at[b].at[j]], sems[b][1],
                                    add=True)
                   for j in range(K2)]
            for dd in sds:
                dd.wait()

        @pl.when(nfull > 0)
        def _():
            fire(0, base)

        def pair_body(i, carry):
            fire(1, base + (2 * i + 1) * K2)
            drain_and_scatter(0)

            @pl.when(2 * i + 2 < nfull)
            def _():
                fire(0, base + (2 * i + 2) * K2)

            drain_and_scatter(1)
            return carry

        lax.fori_loop(0, npair, pair_body, 0)

        @pl.when(nfull - 2 * npair == 1)
        def _():
            drain_and_scatter(0)

        def rem_body(j, carry):
            row = base + nfull * K2 + j
            pltpu.sync_copy(src_h.at[row], srcv.at[0].at[0])
            pltpu.sync_copy(dst_h.at[row], dstv.at[0].at[0])
            pltpu.async_copy(tbl.at[srcv.at[0].at[0]], rowsv.at[0].at[0],
                             gsemA).wait()
            pltpu.sync_copy(rowsv.at[0].at[0], acc.at[dstv.at[0].at[0]],
                            add=True)
            return carry

        lax.fori_loop(0, nrows - nfull * K2, rem_body, 0)

    @pl.when(c == 0)
    def _():
        run(t0_h)

    @pl.when(c == 1)
    def _():
        run(t1_h)

    plsc.subcore_barrier()
    sl = pl.ds(s * TILE_ROWS, TILE_ROWS)
    pltpu.sync_copy(acc.at[sl], out_h.at[c, sl])


@functools.lru_cache(maxsize=None)
def _make_pass(col_split):
    return pl.kernel(
        functools.partial(_pass_body, col_split),
        out_type=jax.ShapeDtypeStruct((N_SC, N_PAD, FW), jnp.float32),
        mesh=_sc_mesh(),
        compiler_params=pltpu.CompilerParams(use_tc_tiling_on_sc=False),
        scratch_types=[
            pltpu.VMEM((2, K2, EL), jnp.int32),
            pltpu.VMEM((2, K2, EL), jnp.int32),
            pltpu.VMEM((2, K2, EL, FW), jnp.float32),
            pltpu.VMEM_SHARED((N_PAD, FW), jnp.float32),
            pltpu.SemaphoreType.DMA,
            pltpu.SemaphoreType.DMA,
            pltpu.SemaphoreType.DMA,
            pltpu.SemaphoreType.DMA,
        ],
    )


BN = 4000
_GRID = N_NODES // BN


def _row_spec(w):
    return pl.BlockSpec((BN, w), lambda i: (i, 0))


def _full_spec(shape):
    return pl.BlockSpec(shape, lambda i: (0, 0))


def _tc0_body(d0, d1, xp, dis, t1):
    deg = d0[...] + d1[...] + 1.0
    dv = lax.rsqrt(deg)
    dis[...] = dv
    t1[...] = dv * xp[...]


_tc0 = pl.pallas_call(
    _tc0_body,
    grid=(_GRID,),
    in_specs=[_row_spec(1), _row_spec(1), _row_spec(FW)],
    out_specs=[_row_spec(1), _row_spec(FW)],
    out_shape=[jax.ShapeDtypeStruct((N_NODES, 1), jnp.float32),
               jax.ShapeDtypeStruct((N_NODES, FW), jnp.float32)],
)


def _tc1_body(dis, p0, p1, t1, w1, b1, w2, t2a, t2b):
    d = dis[...]
    z = d * (p0[...] + p1[...] + t1[...])
    h1 = jnp.maximum(
        jnp.dot(z, w1[...], preferred_element_type=jnp.float32) + b1[...], 0.0)
    t2 = d * jnp.dot(h1, w2[...], preferred_element_type=jnp.float32)
    t2a[...] = t2[:, :FW]
    t2b[...] = t2[:, FW:]


_tc1 = pl.pallas_call(
    _tc1_body,
    grid=(_GRID,),
    in_specs=[_row_spec(1), _row_spec(FW), _row_spec(FW), _row_spec(FW),
              _full_spec((FW, 64)), _full_spec((1, 64)), _full_spec((64, 32))],
    out_specs=[_row_spec(FW), _row_spec(FW)],
    out_shape=[jax.ShapeDtypeStruct((N_NODES, FW), jnp.float32),
               jax.ShapeDtypeStruct((N_NODES, FW), jnp.float32)],
)


def _tc2_body(dis, q0, q1, t2a, t2b, g2, b2, w3, t3):
    d = dis[...]
    za = d * (q0[...] + t2a[...])
    zb = d * (q1[...] + t2b[...])
    z = jnp.concatenate([za, zb], axis=1)
    h2 = jnp.maximum(z * g2[...] + b2[...], 0.0)
    t3[...] = d * jnp.dot(h2, w3[...], preferred_element_type=jnp.float32)


_tc2 = pl.pallas_call(
    _tc2_body,
    grid=(_GRID,),
    in_specs=[_row_spec(1), _row_spec(FW), _row_spec(FW), _row_spec(FW),
              _row_spec(FW), _full_spec((1, 32)), _full_spec((1, 32)),
              _full_spec((32, FW))],
    out_specs=_row_spec(FW),
    out_shape=jax.ShapeDtypeStruct((N_NODES, FW), jnp.float32),
)


def _tc3_body(dis, p0, p1, t3, b3, w4, t4):
    d = dis[...]
    h3 = jnp.maximum(d * (p0[...] + p1[...] + t3[...]) + b3[...], 0.0)
    t4[...] = d * jnp.dot(h3, w4[...], preferred_element_type=jnp.float32)


_tc3 = pl.pallas_call(
    _tc3_body,
    grid=(_GRID,),
    in_specs=[_row_spec(1), _row_spec(FW), _row_spec(FW), _row_spec(FW),
              _full_spec((1, FW)), _full_spec((FW, FW))],
    out_specs=_row_spec(FW),
    out_shape=jax.ShapeDtypeStruct((N_NODES, FW), jnp.float32),
)


def _tc4_body(dis, p0, p1, t4, b4, out):
    out[...] = dis[...] * (p0[...] + p1[...] + t4[...]) + b4[...]


_tc4 = pl.pallas_call(
    _tc4_body,
    grid=(_GRID,),
    in_specs=[_row_spec(1), _row_spec(FW), _row_spec(FW), _row_spec(FW),
              _full_spec((1, FW))],
    out_specs=_row_spec(FW),
    out_shape=jax.ShapeDtypeStruct((N_NODES, FW), jnp.float32),
)


def kernel(x, edge_index, W1, b1, g1, bb1, W2, b2, g2, bb2, W3, b3, W4, b4):
    src2 = edge_index[0].reshape(R_ROWS, EL)
    dst2 = edge_index[1].reshape(R_ROWS, EL)
    ones = jnp.ones((EL,), jnp.float32)
    zeros1 = jnp.zeros((TILE_ROWS,), jnp.float32)
    zeros2 = jnp.zeros((TILE_ROWS, FW), jnp.float32)

    degp = _deg_call()(dst2, ones, zeros1)
    d0 = degp[0, :N_NODES, None]
    d1 = degp[1, :N_NODES, None]
    xp = jnp.pad(x, ((0, 0), (0, FW - x.shape[1])))
    dis, t1 = _tc0(d0, d1, xp)

    p = _make_pass(False)(src2, dst2, t1, t1, zeros2)
    cbn = (1.0 / jnp.sqrt(1.0 + 1e-5)).astype(jnp.float32)
    W1e = jnp.pad(W1, ((0, FW - W1.shape[0]), (0, 0))) * (cbn * g1)[None, :]
    b1e = (b1 * cbn * g1 + bb1)[None, :]
    t2a, t2b = _tc1(dis, p[0, :N_NODES], p[1, :N_NODES], t1, W1e, b1e, W2)

    q = _make_pass(True)(src2, dst2, t2a, t2b, zeros2)
    g2v = (cbn * g2)[None, :]
    b2e = (b2 * cbn * g2 + bb2)[None, :]
    t3 = _tc2(dis, q[0, :N_NODES], q[1, :N_NODES], t2a, t2b, g2v, b2e, W3)

    p = _make_pass(False)(src2, dst2, t3, t3, zeros2)
    W4p = jnp.pad(W4, ((0, 0), (0, FW - W4.shape[1])))
    t4 = _tc3(dis, p[0, :N_NODES], p[1, :N_NODES], t3, b3[None, :], W4p)

    p = _make_pass(False)(src2, dst2, t4, t4, zeros2)
    b4p = jnp.pad(b4, (0, FW - b4.shape[0]))[None, :]
    out16 = _tc4(dis, p[0, :N_NODES], p[1, :N_NODES], t4, b4p)
    return out16[:, :W4.shape[1]]


# speedup vs baseline: 83.3602x; 1.3716x over previous
import functools

import jax
import jax.numpy as jnp
from jax import lax
from jax.experimental import pallas as pl
from jax.experimental.pallas import tpu as pltpu
from jax.experimental.pallas import tpu_sc as plsc

N_NODES = 100000
N_PAD = 102400
E_EDGES = 6400000
EL = 128
R_ROWS = E_EDGES // EL
N_SC = 2
N_TILES = 16
N_WORKERS = N_SC * N_TILES
TILE_ROWS = N_PAD // N_TILES
RB = 800
FW = 16

@functools.lru_cache(maxsize=None)
def _sc_mesh():
    return plsc.VectorSubcoreMesh(
        core_axis_name="c", subcore_axis_name="s", num_cores=N_SC,
        num_subcores=N_TILES)


def _worker_range(wid, total, workers):
    q, r = total // workers, total % workers
    nrows = jnp.where(wid < r, q + 1, q)
    base = wid * q + jnp.minimum(wid, r)
    return base, nrows


K = 24


def _deg_body(edge_h, ones_h, zeros_h, out_h, onesv, dstv, acc, dbuf, bbuf,
              ssem):
    c = lax.axis_index("c")
    s = lax.axis_index("s")
    wid = c * N_TILES + s
    pltpu.sync_copy(ones_h, onesv)
    pltpu.sync_copy(zeros_h, dbuf)
    pltpu.sync_copy(dbuf, acc.at[pl.ds(s * TILE_ROWS, TILE_ROWS)])
    plsc.subcore_barrier()
    base, nrows = _worker_range(wid, R_ROWS, N_WORKERS)
    nfull = nrows // K

    def chunk_body(ci, carry):
        cb = base + ci * K
        pltpu.sync_copy(edge_h.at[1, pl.ds(cb, K)], dstv)
        descs = [pltpu.async_copy(onesv, acc.at[dstv.at[j]], ssem, add=True)
                 for j in range(K)]
        for dd in descs:
            dd.wait()
        return carry

    lax.fori_loop(0, nfull, chunk_body, 0)

    def rem_body(j, carry):
        pltpu.sync_copy(edge_h.at[1, base + nfull * K + j], dstv.at[0])
        pltpu.sync_copy(onesv, acc.at[dstv.at[0]], add=True)
        return carry

    lax.fori_loop(0, nrows - nfull * K, rem_body, 0)
    plsc.subcore_barrier()
    sl = pl.ds(s * TILE_ROWS, TILE_ROWS)
    pltpu.sync_copy(acc.at[sl], dbuf)

    def bcast_body(i, carry):
        v = dbuf[pl.ds(i * 16, 16)]
        for j in range(16):
            bbuf[i * 16 + j] = jnp.broadcast_to(v[j], (16,))
        return carry

    lax.fori_loop(0, TILE_ROWS // 16, bcast_body, 0)
    pltpu.sync_copy(bbuf, out_h.at[c, sl])


@functools.lru_cache(maxsize=None)
def _deg_call():
    return pl.kernel(
        _deg_body,
        out_type=jax.ShapeDtypeStruct((N_SC, N_PAD, FW), jnp.float32),
        mesh=_sc_mesh(),
        compiler_params=pltpu.CompilerParams(use_tc_tiling_on_sc=False),
        scratch_types=[
            pltpu.VMEM((EL,), jnp.float32),
            pltpu.VMEM((K, EL), jnp.int32),
            pltpu.VMEM_SHARED((N_PAD,), jnp.float32),
            pltpu.VMEM((TILE_ROWS,), jnp.float32),
            pltpu.VMEM((TILE_ROWS, FW), jnp.float32),
            pltpu.SemaphoreType.DMA,
        ],
    )


K2 = 6


def _pass_body(col_split, edge_h, t0_h, t1_h, zeros_h, out_h,
               srcv, dstv, rowsv, acc, gsemA, gsemB, ssemA, ssemB):
    c = lax.axis_index("c")
    s = lax.axis_index("s")
    pltpu.sync_copy(zeros_h, acc.at[pl.ds(s * TILE_ROWS, TILE_ROWS)])
    plsc.subcore_barrier()

    if col_split:
        base = s * (R_ROWS // N_TILES)
        nrows = R_ROWS // N_TILES
    else:
        base, nrows = _worker_range(c * N_TILES + s, R_ROWS, N_WORKERS)
    nfull = nrows // K2
    npair = nfull // 2

    def run(tbl):
        sems = [(gsemA, ssemA), (gsemB, ssemB)]

        def fire(b, cb):
            pltpu.sync_copy(edge_h.at[0, pl.ds(cb, K2)], srcv.at[b])
            pltpu.sync_copy(edge_h.at[1, pl.ds(cb, K2)], dstv.at[b])
            for j in range(K2):
                pltpu.async_copy(tbl.at[srcv.at[b].at[j]],
                                 rowsv.at[b].at[j], sems[b][0])

        def drain_and_scatter(b):
            for j in range(K2):
                pltpu.make_async_copy(tbl.at[srcv.at[b].at[j]],
                                      rowsv.at[b].at[j], sems[b][0]).wait()
            sds = [pltpu.async_copy(rowsv.at[b].at[j],
                                    acc.at[dstv.at[b].at[j]], sems[b][1],
                                    add=True)
                   for j in range(K2)]
            for dd in sds:
                dd.wait()

        @pl.when(nfull > 0)
        def _():
            fire(0, base)

        def pair_body(i, carry):
            fire(1, base + (2 * i + 1) * K2)
            drain_and_scatter(0)

            @pl.when(2 * i + 2 < nfull)
            def _():
                fire(0, base + (2 * i + 2) * K2)

            drain_and_scatter(1)
            return carry

        lax.fori_loop(0, npair, pair_body, 0)

        @pl.when(nfull - 2 * npair == 1)
        def _():
            drain_and_scatter(0)

        def rem_body(j, carry):
            row = base + nfull * K2 + j
            pltpu.sync_copy(edge_h.at[0, row], srcv.at[0].at[0])
            pltpu.sync_copy(edge_h.at[1, row], dstv.at[0].at[0])
            pltpu.async_copy(tbl.at[srcv.at[0].at[0]], rowsv.at[0].at[0],
                             gsemA).wait()
            pltpu.sync_copy(rowsv.at[0].at[0], acc.at[dstv.at[0].at[0]],
                            add=True)
            return carry

        lax.fori_loop(0, nrows - nfull * K2, rem_body, 0)

    @pl.when(c == 0)
    def _():
        run(t0_h)

    @pl.when(c == 1)
    def _():
        run(t1_h)

    plsc.subcore_barrier()
    sl = pl.ds(s * TILE_ROWS, TILE_ROWS)
    pltpu.sync_copy(acc.at[sl], out_h.at[c, sl])


@functools.lru_cache(maxsize=None)
def _make_pass(col_split):
    return pl.kernel(
        functools.partial(_pass_body, col_split),
        out_type=jax.ShapeDtypeStruct((N_SC, N_PAD, FW), jnp.float32),
        mesh=_sc_mesh(),
        compiler_params=pltpu.CompilerParams(use_tc_tiling_on_sc=False),
        scratch_types=[
            pltpu.VMEM((2, K2, EL), jnp.int32),
            pltpu.VMEM((2, K2, EL), jnp.int32),
            pltpu.VMEM((2, K2, EL, FW), jnp.float32),
            pltpu.VMEM_SHARED((N_PAD, FW), jnp.float32),
            pltpu.SemaphoreType.DMA,
            pltpu.SemaphoreType.DMA,
            pltpu.SemaphoreType.DMA,
            pltpu.SemaphoreType.DMA,
        ],
    )


BN = 6400
BNP = BN // 8
_GRID = N_PAD // BN
NPK = N_PAD // 8


def _pk_spec():
    return pl.BlockSpec((BNP, EL), lambda i: (i, 0))


def _part_spec(c):
    return pl.BlockSpec((1, BNP, EL), lambda i, c=c: (c, i, 0))


def _full_spec(shape):
    return pl.BlockSpec(shape, lambda i: (0, 0))


def _pk_struct():
    return jax.ShapeDtypeStruct((NPK, EL), jnp.float32)


def _tc0_body(q0, q1, xpk, dis, t1):
    deg = q0[...][0] + q1[...][0] + 1.0
    dv = lax.rsqrt(deg)
    dis[...] = dv
    t1[...] = dv * xpk[...]


_tc0 = pl.pallas_call(
    _tc0_body,
    grid=(_GRID,),
    in_specs=[_part_spec(0), _part_spec(1), _pk_spec()],
    out_specs=[_pk_spec(), _pk_spec()],
    out_shape=[_pk_struct(), _pk_struct()],
)


def _tc1_body(dis, p0, p1, t1, w1, b1, w2a, w2b, t2a, t2b):
    d = dis[...]
    z = d * (p0[...][0] + p1[...][0] + t1[...])
    h1 = jnp.maximum(
        jnp.dot(z, w1[...], preferred_element_type=jnp.float32) + b1[...], 0.0)
    t2a[...] = d * jnp.dot(h1, w2a[...], preferred_element_type=jnp.float32)
    t2b[...] = d * jnp.dot(h1, w2b[...], preferred_element_type=jnp.float32)


_tc1 = pl.pallas_call(
    _tc1_body,
    grid=(_GRID,),
    in_specs=[_pk_spec(), _part_spec(0), _part_spec(1), _pk_spec(),
              _full_spec((EL, 512)), _full_spec((1, 512)),
              _full_spec((512, EL)), _full_spec((512, EL))],
    out_specs=[_pk_spec(), _pk_spec()],
    out_shape=[_pk_struct(), _pk_struct()],
)


def _tc2_body(dis, q0, q1, t2a, t2b, g2a, g2b, b2a, b2b, w3a, w3b, t3):
    d = dis[...]
    ha = jnp.maximum(d * (q0[...][0] + t2a[...]) * g2a[...] + b2a[...], 0.0)
    hb = jnp.maximum(d * (q1[...][0] + t2b[...]) * g2b[...] + b2b[...], 0.0)
    t3[...] = d * (jnp.dot(ha, w3a[...], preferred_element_type=jnp.float32)
                   + jnp.dot(hb, w3b[...], preferred_element_type=jnp.float32))


_tc2 = pl.pallas_call(
    _tc2_body,
    grid=(_GRID,),
    in_specs=[_pk_spec(), _part_spec(0), _part_spec(1), _pk_spec(),
              _pk_spec(), _full_spec((1, EL)), _full_spec((1, EL)),
              _full_spec((1, EL)), _full_spec((1, EL)),
              _full_spec((EL, EL)), _full_spec((EL, EL))],
    out_specs=_pk_spec(),
    out_shape=_pk_struct(),
)


def _tc3_body(dis, p0, p1, t3, b3, w4, t4):
    d = dis[...]
    h3 = jnp.maximum(d * (p0[...][0] + p1[...][0] + t3[...]) + b3[...], 0.0)
    t4[...] = d * jnp.dot(h3, w4[...], preferred_element_type=jnp.float32)


_tc3 = pl.pallas_call(
    _tc3_body,
    grid=(_GRID,),
    in_specs=[_pk_spec(), _part_spec(0), _part_spec(1), _pk_spec(),
              _full_spec((1, EL)), _full_spec((EL, EL))],
    out_specs=_pk_spec(),
    out_shape=_pk_struct(),
)


def _tc4_body(dis, p0, p1, t4, b4, out):
    out[...] = dis[...] * (p0[...][0] + p1[...][0] + t4[...]) + b4[...]


_tc4 = pl.pallas_call(
    _tc4_body,
    grid=(_GRID,),
    in_specs=[_pk_spec(), _part_spec(0), _part_spec(1), _pk_spec(),
              _full_spec((1, EL))],
    out_specs=_pk_spec(),
    out_shape=_pk_struct(),
)


def _bd(w):
    return jnp.kron(jnp.eye(8, dtype=w.dtype), w)


def _t8(v):
    return jnp.tile(v.reshape(1, -1), (1, 8))


def kernel(x, edge_index, W1, b1, g1, bb1, W2, b2, g2, bb2, W3, b3, W4, b4):
    e3 = edge_index.reshape(2, R_ROWS, EL)
    ones = jnp.ones((EL,), jnp.float32)
    zeros1 = jnp.zeros((TILE_ROWS,), jnp.float32)
    zeros2 = jnp.zeros((TILE_ROWS, FW), jnp.float32)

    def packed(a):
        return a.reshape(N_SC, NPK, EL)

    def table(a):
        return a.reshape(N_PAD, FW)

    xpk = jnp.pad(x, ((0, N_PAD - N_NODES), (0, FW - x.shape[1])))
    xpk = xpk.reshape(NPK, EL)

    degp = packed(_deg_call()(e3, ones, zeros1))
    dis, t1 = _tc0(degp, degp, xpk)

    p = packed(_make_pass(False)(e3, table(t1), table(t1), zeros2))
    cbn = (1.0 / jnp.sqrt(1.0 + 1e-5)).astype(jnp.float32)
    W1e = jnp.pad(W1, ((0, FW - W1.shape[0]), (0, 0))) * (cbn * g1)[None, :]
    b1e = b1 * cbn * g1 + bb1
    t2a, t2b = _tc1(dis, p, p, t1, _bd(W1e), _t8(b1e),
                    _bd(W2[:, :FW]), _bd(W2[:, FW:]))

    q = packed(_make_pass(True)(e3, table(t2a), table(t2b), zeros2))
    t3 = _tc2(dis, q, q, t2a, t2b,
              _t8(cbn * g2[:FW]), _t8(cbn * g2[FW:]),
              _t8(b2[:FW] * cbn * g2[:FW] + bb2[:FW]),
              _t8(b2[FW:] * cbn * g2[FW:] + bb2[FW:]),
              _bd(W3[:FW, :]), _bd(W3[FW:, :]))

    p = packed(_make_pass(False)(e3, table(t3), table(t3), zeros2))
    W4p = jnp.pad(W4, ((0, 0), (0, FW - W4.shape[1])))
    t4 = _tc3(dis, p, p, t3, _t8(b3), _bd(W4p))

    p = packed(_make_pass(False)(e3, table(t4), table(t4), zeros2))
    b4p = jnp.pad(b4, (0, FW - b4.shape[0]))
    out16 = _tc4(dis, p, p, t4, _t8(b4p))
    return out16.reshape(N_PAD, FW)[:N_NODES, :W4.shape[1]]
